# Initial kernel scaffold; baseline (speedup 1.0000x reference)
import jax
import jax.numpy as jnp
from jax.experimental import pallas as pl


def kernel(style_ids, key_ids, timefeel_ids, tempos, structures, style_table, key_table, timefeel_table, tempo_W, tempo_b, structure_W, structure_b, final_W, final_b):
    raise NotImplementedError("write your pallas kernel here")


# speedup vs baseline: 5.1082x; 5.1082x over previous
import functools

import jax
import jax.numpy as jnp
from jax.experimental import pallas as pl

_B = 16384
_BLK = 2048
_NB = _B // _BLK


def _body(sid_ref, kid_ref, tid_ref, tmp_ref, str_ref,
          st_ref, kt_ref, tt_ref, tw_ref, tb_ref, sw_ref, sb_ref,
          fw_ref, fb_ref, out_ref):
    fw = fw_ref[...]
    Ws = fw[:, 0:32]
    Wk = fw[:, 32:48]
    Wt = fw[:, 48:64]
    Wtem = fw[:, 64:80]
    Wstr = fw[:, 80:112]

    f32 = jnp.float32
    SP = jnp.dot(st_ref[...], Ws.T, preferred_element_type=f32)
    KP = jnp.dot(kt_ref[...], Wk.T, preferred_element_type=f32)
    TP = jnp.dot(tt_ref[...], Wt.T, preferred_element_type=f32)
    tW = jnp.dot(tw_ref[...].T, Wtem.T, preferred_element_type=f32)
    sW = jnp.dot(sw_ref[...].T, Wstr.T, preferred_element_type=f32)
    bias = (fb_ref[...]
            + jnp.dot(tb_ref[...], Wtem.T, preferred_element_type=f32)
            + jnp.dot(sb_ref[...], Wstr.T, preferred_element_type=f32))

    sid = sid_ref[0, 0, :]
    kid = kid_ref[0, 0, :]
    tid = tid_ref[0, 0, :]

    def onehot(ids, n):
        return (jax.lax.broadcasted_iota(jnp.int32, (_BLK, n), 1)
                == ids[:, None]).astype(f32)

    out = jnp.dot(onehot(sid, 50), SP, preferred_element_type=f32)
    out += jnp.dot(onehot(kid, 24), KP, preferred_element_type=f32)
    out += jnp.dot(onehot(tid, 20), TP, preferred_element_type=f32)
    out += jnp.dot(tmp_ref[0], tW, preferred_element_type=f32)
    out += jnp.dot(str_ref[0], sW, preferred_element_type=f32)
    out_ref[...] = out + bias


def kernel(style_ids, key_ids, timefeel_ids, tempos, structures,
           style_table, key_table, timefeel_table,
           tempo_W, tempo_b, structure_W, structure_b,
           final_W, final_b):
    sid3 = style_ids.astype(jnp.int32).reshape(_NB, 1, _BLK)
    kid3 = key_ids.astype(jnp.int32).reshape(_NB, 1, _BLK)
    tid3 = timefeel_ids.astype(jnp.int32).reshape(_NB, 1, _BLK)
    tmp3 = tempos.reshape(_NB, _BLK, 1)
    str3 = structures.reshape(_NB, _BLK, 10)
    tb2 = tempo_b.reshape(1, 16)
    sb2 = structure_b.reshape(1, 32)
    fb2 = final_b.reshape(1, 128)

    full = lambda shape: pl.BlockSpec(shape, lambda i: (0,) * len(shape))
    grid_spec = pl.GridSpec(
        grid=(_NB,),
        in_specs=[
            pl.BlockSpec((1, 1, _BLK), lambda i: (i, 0, 0)),
            pl.BlockSpec((1, 1, _BLK), lambda i: (i, 0, 0)),
            pl.BlockSpec((1, 1, _BLK), lambda i: (i, 0, 0)),
            pl.BlockSpec((1, _BLK, 1), lambda i: (i, 0, 0)),
            pl.BlockSpec((1, _BLK, 10), lambda i: (i, 0, 0)),
            full((50, 32)),
            full((24, 16)),
            full((20, 16)),
            full((16, 1)),
            full((1, 16)),
            full((32, 10)),
            full((1, 32)),
            full((128, 112)),
            full((1, 128)),
        ],
        out_specs=pl.BlockSpec((_BLK, 128), lambda i: (i, 0)),
    )
    return pl.pallas_call(
        _body,
        grid_spec=grid_spec,
        out_shape=jax.ShapeDtypeStruct((_B, 128), jnp.float32),
    )(sid3, kid3, tid3, tmp3, str3,
      style_table, key_table, timefeel_table,
      tempo_W, tb2, structure_W, sb2, final_W, fb2)
